# baseline (speedup 1.0000x reference)
import functools
import jax
import jax.numpy as jnp
from jax import lax
from jax.experimental import pallas as pl
from jax.experimental.pallas import tpu as pltpu
from jax.experimental.pallas import tpu_sc as plsc

_LANES = 16


def _make_sc_kernel(bs, h, w, nf):
    hw = h * w
    C = 2 * nf
    NC, NS = 2, 16
    NW = NC * NS
    CH = C // NW
    m = max(h, w)

    mesh = plsc.VectorSubcoreMesh(core_axis_name="c", subcore_axis_name="s")

    @functools.partial(
        pl.kernel,
        out_type=jax.ShapeDtypeStruct((bs * C, hw), jnp.float32),
        mesh=mesh,
        scratch_types=[
            pltpu.VMEM((2, m, nf), jnp.float32),
            pltpu.VMEM((CH, hw), jnp.float32),
            pltpu.SemaphoreType.DMA,
        ],
        compiler_params=pltpu.CompilerParams(
            needs_layout_passes=False, use_tc_tiling_on_sc=True
        ),
    )
    def sc_kernel(col_hbm, row_hbm, out_hbm, tbl_v, block_v, sem):
        wid = lax.axis_index("s") * NC + lax.axis_index("c")
        c0 = wid * CH
        pltpu.sync_copy(col_hbm.at[pl.ds(0, w)], tbl_v.at[0, pl.ds(0, w)])
        pltpu.sync_copy(row_hbm.at[pl.ds(0, h)], tbl_v.at[1, pl.ds(0, h)])
        sel = jnp.where(c0 >= nf, jnp.int32(1), jnp.int32(0))
        cc = lax.rem(c0, jnp.int32(nf))
        sel_vec = lax.broadcast(sel, (_LANES,))
        lane = lax.iota(jnp.int32, _LANES)

        def build(g, carry):
            k = g * _LANES + lane
            idx = jnp.where(sel_vec > 0, lax.div(k, w), lax.rem(k, w))
            for r in range(CH):
                ch = lax.broadcast(cc + r, (_LANES,))
                vals = plsc.load_gather(tbl_v, [sel_vec, idx, ch])
                block_v[r, pl.ds(g * _LANES, _LANES)] = vals
            return carry

        lax.fori_loop(0, hw // _LANES, build, 0)

        copies = [
            pltpu.async_copy(block_v, out_hbm.at[pl.ds(b * C + c0, CH)], sem)
            for b in range(bs)
        ]
        for cp in copies:
            cp.wait()

    return sc_kernel


def kernel(mask, row_embed, col_embed):
    bs = mask.shape[0]
    h, w = mask.shape[-2:]
    nf = row_embed.shape[1]
    out = _make_sc_kernel(bs, h, w, nf)(col_embed, row_embed)
    return out.reshape(bs, 2 * nf, h, w)


# speedup vs baseline: 3.7042x; 3.7042x over previous
import functools
import jax
import jax.numpy as jnp
from jax import lax
from jax.experimental import pallas as pl
from jax.experimental.pallas import tpu as pltpu
from jax.experimental.pallas import tpu_sc as plsc

_LANES = 16


def _make_sc_kernel(bs, h, w, nf):
    C = 2 * nf
    NC, NS = 2, 16
    NW = NC * NS
    assert h == NW, "one y row per vector subcore"

    mesh = plsc.VectorSubcoreMesh(core_axis_name="c", subcore_axis_name="s")

    @functools.partial(
        pl.kernel,
        out_type=jax.ShapeDtypeStruct((bs, h, w, C), jnp.float32),
        mesh=mesh,
        scratch_types=[
            pltpu.VMEM((w, nf), jnp.float32),
            pltpu.VMEM((1, nf), jnp.float32),
            pltpu.VMEM((w, C), jnp.float32),
            pltpu.SemaphoreType.DMA,
        ],
        compiler_params=pltpu.CompilerParams(needs_layout_passes=False),
    )
    def sc_kernel(col_hbm, row_hbm, out_hbm, col_v, row_v, block_v, sem):
        y = lax.axis_index("s") * NC + lax.axis_index("c")
        pltpu.sync_copy(col_hbm.at[pl.ds(0, w)], col_v)
        pltpu.sync_copy(row_hbm.at[pl.ds(y, 1)], row_v)

        def build(x, carry):
            for j in range(nf // _LANES):
                block_v[x, pl.ds(j * _LANES, _LANES)] = col_v[
                    x, pl.ds(j * _LANES, _LANES)
                ]
                block_v[x, pl.ds(nf + j * _LANES, _LANES)] = row_v[
                    0, pl.ds(j * _LANES, _LANES)
                ]
            return carry

        lax.fori_loop(0, w, build, 0)

        copies = [
            pltpu.async_copy(block_v, out_hbm.at[b, y], sem) for b in range(bs)
        ]
        for cp in copies:
            cp.wait()

    return sc_kernel


def kernel(mask, row_embed, col_embed):
    bs = mask.shape[0]
    h, w = mask.shape[-2:]
    nf = row_embed.shape[1]
    out = _make_sc_kernel(bs, h, w, nf)(col_embed, row_embed)
    return out.transpose(0, 3, 1, 2)
